# baseline (speedup 1.0000x reference)
import functools

import jax
import jax.numpy as jnp
from jax import lax
from jax.experimental import pallas as pl
from jax.experimental.pallas import tpu as pltpu
from jax.experimental.pallas import tpu_sc as plsc

_NC = 2
_NS = 16
_NW = _NC * _NS
_K = 128
_CPT = 80
_BM = 2000


def _sc_scatter_partials(h, src2d, dst2d, zeros):
    n, d = h.shape
    npad = zeros.shape[0]
    _, cpw, k = src2d.shape
    rpt = npad // _NS

    def body(h_hbm, src_hbm, dst_hbm, z_hbm, out_hbm, src_v, dst_v, rows_v,
             acc_sh, sem):
        cid = lax.axis_index("c")
        sid = lax.axis_index("s")
        wid = sid * _NC + cid
        pltpu.sync_copy(z_hbm.at[pl.ds(sid * rpt, rpt)],
                        acc_sh.at[pl.ds(sid * rpt, rpt)])
        pltpu.sync_copy(src_hbm.at[wid], src_v)
        pltpu.sync_copy(dst_hbm.at[wid], dst_v)
        plsc.subcore_barrier()

        def step(j, carry):
            pltpu.async_copy(h_hbm.at[src_v.at[j]], rows_v, sem).wait()
            pltpu.sync_copy(rows_v, acc_sh.at[dst_v.at[j]], add=True)
            return carry

        lax.fori_loop(0, cpw, step, 0)
        plsc.subcore_barrier()
        row0 = cid * npad + sid * rpt
        pltpu.sync_copy(acc_sh.at[pl.ds(sid * rpt, rpt)],
                        out_hbm.at[pl.ds(row0, rpt)])

    fn = pl.kernel(
        body,
        out_type=jax.ShapeDtypeStruct((_NC * npad, d), jnp.float32),
        mesh=plsc.VectorSubcoreMesh(core_axis_name="c", subcore_axis_name="s"),
        scratch_types=[
            pltpu.VMEM((cpw, k), jnp.int32),
            pltpu.VMEM((cpw, k), jnp.int32),
            pltpu.VMEM((k, d), jnp.float32),
            pltpu.VMEM_SHARED((npad, d), jnp.float32),
            pltpu.SemaphoreType.DMA,
        ],
    )
    return fn(h, src2d, dst2d, zeros).reshape(_NC, npad, d)


def _mm_kernel(x_ref, w_ref, o_ref):
    o_ref[...] = jnp.dot(x_ref[...], w_ref[...],
                         preferred_element_type=jnp.float32)


def _relu_mm_kernel(p_ref, b_ref, w_ref, o_ref):
    t = jnp.maximum(p_ref[0] + p_ref[1] + b_ref[...], 0.0)
    o_ref[...] = jnp.dot(t, w_ref[...], preferred_element_type=jnp.float32)


def _mm_softmax_kernel(q_ref, b2_ref, wfc_ref, bfc_ref, o_ref):
    t = q_ref[0] + q_ref[1] + b2_ref[...]
    z = jnp.dot(t, wfc_ref[...],
                preferred_element_type=jnp.float32) + bfc_ref[...]
    z = z - jnp.max(z, axis=-1, keepdims=True)
    e = jnp.exp(z)
    o_ref[...] = e / jnp.sum(e, axis=-1, keepdims=True)


def kernel(x, edge_index, W1, b1, W2, b2, Wfc, bfc):
    n, d = x.shape
    e = edge_index.shape[1]
    nclass = Wfc.shape[1]
    align = 8 * _NS
    npad = (n + align - 1) // align * align
    ecap = _NW * _CPT * _K
    assert ecap >= e and npad > n and n % _BM == 0

    pad = ecap - e
    src2d = jnp.concatenate(
        [edge_index[0], jnp.zeros((pad,), jnp.int32)]).reshape(_NW, _CPT, _K)
    dst2d = jnp.concatenate(
        [edge_index[1], jnp.full((pad,), n, jnp.int32)]).reshape(_NW, _CPT, _K)
    zeros = jnp.zeros((npad, d), jnp.float32)

    h1 = pl.pallas_call(
        _mm_kernel,
        grid=(n // _BM,),
        in_specs=[
            pl.BlockSpec((_BM, d), lambda i: (i, 0)),
            pl.BlockSpec((d, d), lambda i: (0, 0)),
        ],
        out_specs=pl.BlockSpec((_BM, d), lambda i: (i, 0)),
        out_shape=jax.ShapeDtypeStruct((n, d), jnp.float32),
    )(x, W1)

    p = _sc_scatter_partials(h1, src2d, dst2d, zeros)
    t = pl.pallas_call(
        _relu_mm_kernel,
        grid=(n // _BM,),
        in_specs=[
            pl.BlockSpec((_NC, _BM, d), lambda i: (0, i, 0)),
            pl.BlockSpec((1, d), lambda i: (0, 0)),
            pl.BlockSpec((d, d), lambda i: (0, 0)),
        ],
        out_specs=pl.BlockSpec((_BM, d), lambda i: (i, 0)),
        out_shape=jax.ShapeDtypeStruct((n, d), jnp.float32),
    )(p, b1.reshape(1, d), W2)

    q = _sc_scatter_partials(t, src2d, dst2d, zeros)
    out = pl.pallas_call(
        _mm_softmax_kernel,
        grid=(n // _BM,),
        in_specs=[
            pl.BlockSpec((_NC, _BM, d), lambda i: (0, i, 0)),
            pl.BlockSpec((1, d), lambda i: (0, 0)),
            pl.BlockSpec((d, nclass), lambda i: (0, 0)),
            pl.BlockSpec((1, nclass), lambda i: (0, 0)),
        ],
        out_specs=pl.BlockSpec((_BM, nclass), lambda i: (i, 0)),
        out_shape=jax.ShapeDtypeStruct((n, nclass), jnp.float32),
    )(q, b2.reshape(1, d), Wfc, bfc.reshape(1, nclass))
    return out


# speedup vs baseline: 2.5119x; 2.5119x over previous
import functools

import jax
import jax.numpy as jnp
from jax import lax
from jax.experimental import pallas as pl
from jax.experimental.pallas import tpu as pltpu
from jax.experimental.pallas import tpu_sc as plsc

_NC = 2
_NS = 16
_NW = _NC * _NS
_K = 100
_BM = 2000


def _sc_scatter_partials(h, src2d, dst2d, zeros):
    n, d = h.shape
    npad = zeros.shape[0]
    _, cpw, k = src2d.shape
    rpt = npad // _NS

    def body(h_hbm, src_hbm, dst_hbm, z_hbm, out_hbm, src_v, dst_v, rows_v,
             acc_sh, sem):
        cid = lax.axis_index("c")
        sid = lax.axis_index("s")
        wid = sid * _NC + cid
        pltpu.sync_copy(z_hbm.at[pl.ds(sid * rpt, rpt)],
                        acc_sh.at[pl.ds(sid * rpt, rpt)])
        pltpu.sync_copy(src_hbm.at[wid], src_v)
        pltpu.sync_copy(dst_hbm.at[wid], dst_v)
        plsc.subcore_barrier()

        def step(j, carry):
            pltpu.async_copy(h_hbm.at[src_v.at[j]], rows_v, sem).wait()
            pltpu.sync_copy(rows_v, acc_sh.at[dst_v.at[j]], add=True)
            return carry

        lax.fori_loop(0, cpw, step, 0)
        plsc.subcore_barrier()
        row0 = cid * npad + sid * rpt
        pltpu.sync_copy(acc_sh.at[pl.ds(sid * rpt, rpt)],
                        out_hbm.at[pl.ds(row0, rpt)])

    fn = pl.kernel(
        body,
        out_type=jax.ShapeDtypeStruct((_NC * npad, d), jnp.float32),
        mesh=plsc.VectorSubcoreMesh(core_axis_name="c", subcore_axis_name="s"),
        scratch_types=[
            pltpu.VMEM((cpw, k), jnp.int32),
            pltpu.VMEM((cpw, k), jnp.int32),
            pltpu.VMEM((k, d), jnp.float32),
            pltpu.VMEM_SHARED((npad, d), jnp.float32),
            pltpu.SemaphoreType.DMA,
        ],
    )
    return fn(h, src2d, dst2d, zeros).reshape(_NC, npad, d)


def _mm_kernel(x_ref, w_ref, o_ref):
    o_ref[...] = jnp.dot(x_ref[...], w_ref[...],
                         preferred_element_type=jnp.float32)


def _relu_mm_kernel(p_ref, b_ref, w_ref, o_ref):
    t = jnp.maximum(p_ref[0] + p_ref[1] + b_ref[...], 0.0)
    o_ref[...] = jnp.dot(t, w_ref[...], preferred_element_type=jnp.float32)


def _mm_softmax_kernel(q_ref, b2_ref, wfc_ref, bfc_ref, o_ref):
    t = q_ref[0] + q_ref[1] + b2_ref[...]
    z = jnp.dot(t, wfc_ref[...],
                preferred_element_type=jnp.float32) + bfc_ref[...]
    z = z - jnp.max(z, axis=-1, keepdims=True)
    e = jnp.exp(z)
    o_ref[...] = e / jnp.sum(e, axis=-1, keepdims=True)


def kernel(x, edge_index, W1, b1, W2, b2, Wfc, bfc):
    n, d = x.shape
    e = edge_index.shape[1]
    nclass = Wfc.shape[1]
    align = 8 * _NS
    npad = (n + align - 1) // align * align
    assert e % (_NW * _K) == 0 and npad > n and n % _BM == 0

    src2d = edge_index[0].reshape(_NW, e // (_NW * _K), _K)
    dst2d = edge_index[1].reshape(_NW, e // (_NW * _K), _K)
    zeros = jnp.zeros((npad, d), jnp.float32)

    h1 = pl.pallas_call(
        _mm_kernel,
        grid=(n // _BM,),
        in_specs=[
            pl.BlockSpec((_BM, d), lambda i: (i, 0)),
            pl.BlockSpec((d, d), lambda i: (0, 0)),
        ],
        out_specs=pl.BlockSpec((_BM, d), lambda i: (i, 0)),
        out_shape=jax.ShapeDtypeStruct((n, d), jnp.float32),
    )(x, W1)

    p = _sc_scatter_partials(h1, src2d, dst2d, zeros)
    t = pl.pallas_call(
        _relu_mm_kernel,
        grid=(n // _BM,),
        in_specs=[
            pl.BlockSpec((_NC, _BM, d), lambda i: (0, i, 0)),
            pl.BlockSpec((1, d), lambda i: (0, 0)),
            pl.BlockSpec((d, d), lambda i: (0, 0)),
        ],
        out_specs=pl.BlockSpec((_BM, d), lambda i: (i, 0)),
        out_shape=jax.ShapeDtypeStruct((n, d), jnp.float32),
    )(p, b1.reshape(1, d), W2)

    q = _sc_scatter_partials(t, src2d, dst2d, zeros)
    out = pl.pallas_call(
        _mm_softmax_kernel,
        grid=(n // _BM,),
        in_specs=[
            pl.BlockSpec((_NC, _BM, d), lambda i: (0, i, 0)),
            pl.BlockSpec((1, d), lambda i: (0, 0)),
            pl.BlockSpec((d, nclass), lambda i: (0, 0)),
            pl.BlockSpec((1, nclass), lambda i: (0, 0)),
        ],
        out_specs=pl.BlockSpec((_BM, nclass), lambda i: (i, 0)),
        out_shape=jax.ShapeDtypeStruct((n, nclass), jnp.float32),
    )(q, b2.reshape(1, d), Wfc, bfc.reshape(1, nclass))
    return out


# speedup vs baseline: 2.6680x; 1.0622x over previous
import functools

import jax
import jax.numpy as jnp
from jax import lax
from jax.experimental import pallas as pl
from jax.experimental.pallas import tpu as pltpu
from jax.experimental.pallas import tpu_sc as plsc

_NC = 2
_NS = 16
_NW = _NC * _NS
_K = 125
_BM = 2000


def _sc_scatter_partials(h, src2d, dst2d, zeros):
    n, d = h.shape
    npad = zeros.shape[0]
    _, cpw, k = src2d.shape
    rpt = npad // _NS

    def body(h_hbm, src_hbm, dst_hbm, z_hbm, out_hbm, src_v, dst_v, rows_v,
             acc_sh, sem):
        cid = lax.axis_index("c")
        sid = lax.axis_index("s")
        wid = sid * _NC + cid
        pltpu.sync_copy(z_hbm.at[pl.ds(sid * rpt, rpt)],
                        acc_sh.at[pl.ds(sid * rpt, rpt)])
        pltpu.sync_copy(src_hbm.at[wid], src_v)
        pltpu.sync_copy(dst_hbm.at[wid], dst_v)
        plsc.subcore_barrier()

        def step(j, carry):
            pltpu.async_copy(h_hbm.at[src_v.at[j]], rows_v, sem).wait()
            pltpu.sync_copy(rows_v, acc_sh.at[dst_v.at[j]], add=True)
            return carry

        lax.fori_loop(0, cpw, step, 0)
        plsc.subcore_barrier()
        row0 = cid * npad + sid * rpt
        pltpu.sync_copy(acc_sh.at[pl.ds(sid * rpt, rpt)],
                        out_hbm.at[pl.ds(row0, rpt)])

    fn = pl.kernel(
        body,
        out_type=jax.ShapeDtypeStruct((_NC * npad, d), jnp.float32),
        mesh=plsc.VectorSubcoreMesh(core_axis_name="c", subcore_axis_name="s"),
        scratch_types=[
            pltpu.VMEM((cpw, k), jnp.int32),
            pltpu.VMEM((cpw, k), jnp.int32),
            pltpu.VMEM((k, d), jnp.float32),
            pltpu.VMEM_SHARED((npad, d), jnp.float32),
            pltpu.SemaphoreType.DMA,
        ],
    )
    return fn(h, src2d, dst2d, zeros).reshape(_NC, npad, d)


def _mm_kernel(x_ref, w_ref, o_ref):
    o_ref[...] = jnp.dot(x_ref[...], w_ref[...],
                         preferred_element_type=jnp.float32)


def _relu_mm_kernel(p_ref, b_ref, w_ref, o_ref):
    t = jnp.maximum(p_ref[0] + p_ref[1] + b_ref[...], 0.0)
    o_ref[...] = jnp.dot(t, w_ref[...], preferred_element_type=jnp.float32)


def _mm_softmax_kernel(q_ref, b2_ref, wfc_ref, bfc_ref, o_ref):
    t = q_ref[0] + q_ref[1] + b2_ref[...]
    z = jnp.dot(t, wfc_ref[...],
                preferred_element_type=jnp.float32) + bfc_ref[...]
    z = z - jnp.max(z, axis=-1, keepdims=True)
    e = jnp.exp(z)
    o_ref[...] = e / jnp.sum(e, axis=-1, keepdims=True)


def kernel(x, edge_index, W1, b1, W2, b2, Wfc, bfc):
    n, d = x.shape
    e = edge_index.shape[1]
    nclass = Wfc.shape[1]
    align = 8 * _NS
    npad = (n + align - 1) // align * align
    assert e % (_NW * _K) == 0 and npad > n and n % _BM == 0

    src2d = edge_index[0].reshape(_NW, e // (_NW * _K), _K)
    dst2d = edge_index[1].reshape(_NW, e // (_NW * _K), _K)
    zeros = jnp.zeros((npad, d), jnp.float32)

    h1 = pl.pallas_call(
        _mm_kernel,
        grid=(n // _BM,),
        in_specs=[
            pl.BlockSpec((_BM, d), lambda i: (i, 0)),
            pl.BlockSpec((d, d), lambda i: (0, 0)),
        ],
        out_specs=pl.BlockSpec((_BM, d), lambda i: (i, 0)),
        out_shape=jax.ShapeDtypeStruct((n, d), jnp.float32),
    )(x, W1)

    p = _sc_scatter_partials(h1, src2d, dst2d, zeros)
    t = pl.pallas_call(
        _relu_mm_kernel,
        grid=(n // _BM,),
        in_specs=[
            pl.BlockSpec((_NC, _BM, d), lambda i: (0, i, 0)),
            pl.BlockSpec((1, d), lambda i: (0, 0)),
            pl.BlockSpec((d, d), lambda i: (0, 0)),
        ],
        out_specs=pl.BlockSpec((_BM, d), lambda i: (i, 0)),
        out_shape=jax.ShapeDtypeStruct((n, d), jnp.float32),
    )(p, b1.reshape(1, d), W2)

    q = _sc_scatter_partials(t, src2d, dst2d, zeros)
    out = pl.pallas_call(
        _mm_softmax_kernel,
        grid=(n // _BM,),
        in_specs=[
            pl.BlockSpec((_NC, _BM, d), lambda i: (0, i, 0)),
            pl.BlockSpec((1, d), lambda i: (0, 0)),
            pl.BlockSpec((d, nclass), lambda i: (0, 0)),
            pl.BlockSpec((1, nclass), lambda i: (0, 0)),
        ],
        out_specs=pl.BlockSpec((_BM, nclass), lambda i: (i, 0)),
        out_shape=jax.ShapeDtypeStruct((n, nclass), jnp.float32),
    )(q, b2.reshape(1, d), Wfc, bfc.reshape(1, nclass))
    return out
